# baseline (speedup 1.0000x reference)
import functools

import jax
import jax.numpy as jnp
from jax import lax
from jax.experimental import pallas as pl
from jax.experimental.pallas import tpu as pltpu

B, D, S = 4, 1024, 2048
K = 256
_ROWS_PER_STEP = 1024
_G, _L = 16, 128


def _mean_kernel(w_ref, out_ref, acc_ref):
    j = pl.program_id(1)
    nj = pl.num_programs(1)

    @pl.when(j == 0)
    def _init():
        acc_ref[...] = jnp.zeros_like(acc_ref)

    acc = acc_ref[...]
    for g in range(_ROWS_PER_STEP // 8):
        acc = acc + w_ref[0, 8 * g:8 * g + 8, :]
    acc_ref[...] = acc

    @pl.when(j == nj - 1)
    def _finish():
        a = acc_ref[...]
        t = a[0:4] + a[4:8]
        u = t[0:2] + t[2:4]
        s = u[0:1] + u[1:2]
        out_ref[0] = s * (1.0 / S)


def _topk_kernel(m_ref, idx_ref):
    v = m_ref[...].reshape(B, _G, _L)
    jg = lax.broadcasted_iota(jnp.int32, (B, _G, _L), 1)
    jl = lax.broadcasted_iota(jnp.int32, (B, _G, _L), 2)
    i = jg * _L + jl

    def cmpex(v, i, d, k):
        if d >= _L:
            dd = d // _L
            is_lo = (jg & dd) == 0
            vp = jnp.where(is_lo, jnp.roll(v, -dd, axis=1),
                           jnp.roll(v, dd, axis=1))
            ip = jnp.where(is_lo, jnp.roll(i, -dd, axis=1),
                           jnp.roll(i, dd, axis=1))
        else:
            is_lo = (jl & d) == 0
            vp = jnp.where(is_lo, jnp.roll(v, -d, axis=2),
                           jnp.roll(v, d, axis=2))
            ip = jnp.where(is_lo, jnp.roll(i, -d, axis=2),
                           jnp.roll(i, d, axis=2))
        if k >= S:
            asc = jnp.full(v.shape, True)
        elif k >= _L:
            asc = (jg & (k // _L)) == 0
        else:
            asc = (jl & k) == 0
        own_first = (v > vp) | ((v == vp) & (i < ip))
        take_own = own_first == (is_lo == asc)
        return jnp.where(take_own, v, vp), jnp.where(take_own, i, ip)

    k = 2
    while k <= S:
        d = k // 2
        while d >= 1:
            v, i = cmpex(v, i, d, k)
            d //= 2
        k *= 2

    idx_ref[...] = i[:, 0:K // _L, :].reshape(B, K)


def _gather_kernel(x_ref, idx_ref, out_ref):
    idx_row = idx_ref[0]
    onehot = (lax.broadcasted_iota(jnp.int32, (S, K), 0)
              == idx_row).astype(jnp.bfloat16)
    xb = x_ref[0]
    mask = jnp.uint32(0xFFFF0000)
    hi_f = lax.bitcast_convert_type(
        lax.bitcast_convert_type(xb, jnp.uint32) & mask, jnp.float32)
    r = xb - hi_f
    mid_f = lax.bitcast_convert_type(
        lax.bitcast_convert_type(r, jnp.uint32) & mask, jnp.float32)
    lo_f = r - mid_f
    acc = jnp.dot(hi_f.astype(jnp.bfloat16), onehot,
                  preferred_element_type=jnp.float32)
    acc = acc + jnp.dot(mid_f.astype(jnp.bfloat16), onehot,
                        preferred_element_type=jnp.float32)
    acc = acc + jnp.dot(lo_f.astype(jnp.bfloat16), onehot,
                        preferred_element_type=jnp.float32)
    out_ref[0] = acc


@functools.partial(jax.jit)
def kernel(x, w):
    nsteps = S // _ROWS_PER_STEP
    w_mean = pl.pallas_call(
        _mean_kernel,
        grid=(B, nsteps),
        in_specs=[pl.BlockSpec((1, _ROWS_PER_STEP, S),
                               lambda b, j: (b, j, 0))],
        out_specs=pl.BlockSpec((1, 1, S), lambda b, j: (b, 0, 0)),
        out_shape=jax.ShapeDtypeStruct((B, 1, S), jnp.float32),
        scratch_shapes=[pltpu.VMEM((8, S), jnp.float32)],
        compiler_params=pltpu.CompilerParams(
            dimension_semantics=("arbitrary", "arbitrary")),
    )(w)

    idx = pl.pallas_call(
        _topk_kernel,
        out_shape=jax.ShapeDtypeStruct((B, K), jnp.int32),
    )(w_mean.reshape(B, S))

    idx3 = idx.reshape(B, 1, K)
    out = pl.pallas_call(
        _gather_kernel,
        grid=(B,),
        in_specs=[
            pl.BlockSpec((1, D, S), lambda b: (b, 0, 0)),
            pl.BlockSpec((1, 1, K), lambda b: (b, 0, 0)),
        ],
        out_specs=pl.BlockSpec((1, D, K), lambda b: (b, 0, 0)),
        out_shape=jax.ShapeDtypeStruct((B, D, K), jnp.float32),
        compiler_params=pltpu.CompilerParams(
            dimension_semantics=("arbitrary",)),
    )(x, idx3)
    return out
